# Initial kernel scaffold; baseline (speedup 1.0000x reference)
import jax
import jax.numpy as jnp
from jax.experimental import pallas as pl


def kernel(boxes, classification, detections):
    raise NotImplementedError("write your pallas kernel here")


# speedup vs baseline: 6.1451x; 6.1451x over previous
import functools

import jax
import jax.numpy as jnp
from jax import lax
from jax.experimental import pallas as pl
from jax.experimental.pallas import tpu as pltpu
from jax.experimental.pallas import tpu_sc as plsc

_NUM_CLASSES = 80
_THR = 0.4
_K = 1000
_KPAD = 1024
_MAXB = 300
_MPAD = 304
_N = 20000
_DDIM = 84
_DPAD = 128


def _smax_body(c_ref, o_ref):
    o_ref[...] = jnp.max(c_ref[...], axis=3)


def _scores(classification):
    c4 = classification.reshape(10, _N // 80, 8, _NUM_CLASSES)
    out = pl.pallas_call(
        _smax_body,
        grid=(10,),
        in_specs=[pl.BlockSpec((1, _N // 80, 8, _NUM_CLASSES),
                               lambda i: (i, 0, 0, 0))],
        out_specs=pl.BlockSpec((1, _N // 80, 8), lambda i: (i, 0, 0)),
        out_shape=jax.ShapeDtypeStruct((10, _N // 80, 8), jnp.float32),
    )(c4)
    return out.reshape(_N)


def _make_sc_gather():
    info = plsc.get_sparse_core_info()
    nc, ns = info.num_cores, info.num_subcores
    nw = nc * ns
    bpw = _KPAD // nw
    mesh = plsc.VectorSubcoreMesh(core_axis_name="c", subcore_axis_name="s")

    @functools.partial(
        pl.kernel,
        mesh=mesh,
        out_type=jax.ShapeDtypeStruct((_KPAD, _DPAD), jnp.float32),
        scratch_types=[
            pltpu.VMEM((bpw,), jnp.int32),
            pltpu.VMEM((bpw, _DPAD), jnp.float32),
            pltpu.SemaphoreType.DMA,
        ],
    )
    def gk(table_hbm, idx_hbm, out_hbm, idx_v, rows_v, sem):
        wid = lax.axis_index("s") * nc + lax.axis_index("c")
        base = wid * bpw
        pltpu.sync_copy(idx_hbm.at[pl.ds(base, bpw)], idx_v)
        pltpu.async_copy(table_hbm.at[idx_v], rows_v, sem).wait()
        pltpu.sync_copy(rows_v, out_hbm.at[pl.ds(base, bpw)])

    return gk


def _nms_body(g_ref, bT_ref, o_ref, s_scr):
    bT = bT_ref[...]
    x1r = bT[0:1, :]
    y1r = bT[1:2, :]
    x2r = bT[2:3, :]
    y2r = bT[3:4, :]
    area_r = jnp.maximum(x2r - x1r, 0.0) * jnp.maximum(y2r - y1r, 0.0)
    col_j = lax.broadcasted_iota(jnp.int32, (32, _KPAD), 1)
    for c in range(_KPAD // 32):
        cb = g_ref[c * 32:(c + 1) * 32, 0:4]
        x1c = cb[:, 0:1]
        y1c = cb[:, 1:2]
        x2c = cb[:, 2:3]
        y2c = cb[:, 3:4]
        area_c = jnp.maximum(x2c - x1c, 0.0) * jnp.maximum(y2c - y1c, 0.0)
        ix1 = jnp.maximum(x1c, x1r)
        iy1 = jnp.maximum(y1c, y1r)
        ix2 = jnp.minimum(x2c, x2r)
        iy2 = jnp.minimum(y2c, y2r)
        inter = jnp.maximum(ix2 - ix1, 0.0) * jnp.maximum(iy2 - iy1, 0.0)
        union = area_c + area_r - inter
        iou = inter / jnp.maximum(union, 1e-8)
        row_i = lax.broadcasted_iota(jnp.int32, (32, _KPAD), 0) + c * 32
        s_scr[c * 32:(c + 1) * 32, :] = jnp.where(
            (iou > _THR) & (col_j > row_i), 1.0, 0.0)

    lane = lax.broadcasted_iota(jnp.int32, (1, _KPAD), 1)
    a0 = jnp.where(lane < _K, 1.0, 0.0)

    def body(i, a):
        ai = jnp.max(jnp.where(lane == i, a, 0.0))
        sup = s_scr[pl.ds(i, 1), :]
        return a - ai * (a * sup)

    a = lax.fori_loop(0, _K, body, a0)

    ii = lax.broadcasted_iota(jnp.int32, (_KPAD, _KPAD), 0)
    jj = lax.broadcasted_iota(jnp.int32, (_KPAD, _KPAD), 1)
    tri = jnp.where(ii <= jj, 1.0, 0.0)
    rank = lax.dot_general(a, tri, (((1,), (0,)), ((), ()))) - 1.0
    rank_i = rank.astype(jnp.int32)

    rr = lax.broadcasted_iota(jnp.int32, (_MPAD, _KPAD), 0)
    sel = jnp.where((rr == rank_i) & (a > 0.5), 1.0, 0.0)
    o_ref[...] = lax.dot_general(sel, g_ref[...], (((1,), (0,)), ((), ())),
                                 precision=lax.Precision.HIGHEST)


def _nms_call(g, bT):
    return pl.pallas_call(
        _nms_body,
        out_shape=jax.ShapeDtypeStruct((_MPAD, _DPAD), jnp.float32),
        scratch_shapes=[pltpu.VMEM((_KPAD, _KPAD), jnp.float32)],
    )(g, bT)


def kernel(boxes, classification, detections):
    b = boxes[0]
    c = classification[0]
    d = detections[0]
    scores = _scores(c)
    _, idx = lax.top_k(scores, _K)
    idxp = jnp.concatenate(
        [idx.astype(jnp.int32), jnp.zeros((_KPAD - _K,), jnp.int32)])
    table = jnp.concatenate(
        [b, d, jnp.zeros((_N, _DPAD - 4 - _DDIM), jnp.float32)], axis=1)
    g = _make_sc_gather()(table, idxp)
    bT = jnp.pad(g[:, :4].T, ((0, 4), (0, 0)))
    res = _nms_call(g, bT)
    return res[None, :_MAXB, 4:4 + _DDIM]


# speedup vs baseline: 9.1567x; 1.4901x over previous
import functools

import jax
import jax.numpy as jnp
from jax import lax
from jax.experimental import pallas as pl
from jax.experimental.pallas import tpu as pltpu
from jax.experimental.pallas import tpu_sc as plsc

_NUM_CLASSES = 80
_THR = 0.4
_K = 1000
_KPAD = 1024
_MAXB = 300
_MPAD = 304
_N = 20000
_DDIM = 84
_DPAD = 128


def _smax_body(c_ref, o_ref):
    o_ref[...] = jnp.max(c_ref[...], axis=3)


def _scores(classification):
    c4 = classification.reshape(10, _N // 80, 8, _NUM_CLASSES)
    out = pl.pallas_call(
        _smax_body,
        grid=(10,),
        in_specs=[pl.BlockSpec((1, _N // 80, 8, _NUM_CLASSES),
                               lambda i: (i, 0, 0, 0))],
        out_specs=pl.BlockSpec((1, _N // 80, 8), lambda i: (i, 0, 0)),
        out_shape=jax.ShapeDtypeStruct((10, _N // 80, 8), jnp.float32),
    )(c4)
    return out.reshape(_N)


def _make_sc_gather():
    info = plsc.get_sparse_core_info()
    nc, ns = info.num_cores, info.num_subcores
    nw = nc * ns
    bpw = _KPAD // nw
    mesh = plsc.VectorSubcoreMesh(core_axis_name="c", subcore_axis_name="s")

    @functools.partial(
        pl.kernel,
        mesh=mesh,
        out_type=jax.ShapeDtypeStruct((_KPAD, _DPAD), jnp.float32),
        scratch_types=[
            pltpu.VMEM((bpw,), jnp.int32),
            pltpu.VMEM((bpw, _DPAD), jnp.float32),
            pltpu.SemaphoreType.DMA,
        ],
    )
    def gk(table_hbm, idx_hbm, out_hbm, idx_v, rows_v, sem):
        wid = lax.axis_index("s") * nc + lax.axis_index("c")
        base = wid * bpw
        pltpu.sync_copy(idx_hbm.at[pl.ds(base, bpw)], idx_v)
        pltpu.async_copy(table_hbm.at[idx_v], rows_v, sem).wait()
        pltpu.sync_copy(rows_v, out_hbm.at[pl.ds(base, bpw)])

    return gk


def _nms_body(g_ref, bT_ref, o_ref, s_scr):
    bT = bT_ref[...]
    x1r = bT[0:1, :]
    y1r = bT[1:2, :]
    x2r = bT[2:3, :]
    y2r = bT[3:4, :]
    area_r = jnp.maximum(x2r - x1r, 0.0) * jnp.maximum(y2r - y1r, 0.0)
    col_j = lax.broadcasted_iota(jnp.int32, (32, _KPAD), 1)
    for c in range(_KPAD // 32):
        cb = g_ref[c * 32:(c + 1) * 32, 0:4]
        x1c = cb[:, 0:1]
        y1c = cb[:, 1:2]
        x2c = cb[:, 2:3]
        y2c = cb[:, 3:4]
        area_c = jnp.maximum(x2c - x1c, 0.0) * jnp.maximum(y2c - y1c, 0.0)
        ix1 = jnp.maximum(x1c, x1r)
        iy1 = jnp.maximum(y1c, y1r)
        ix2 = jnp.minimum(x2c, x2r)
        iy2 = jnp.minimum(y2c, y2r)
        inter = jnp.maximum(ix2 - ix1, 0.0) * jnp.maximum(iy2 - iy1, 0.0)
        union = area_c + area_r - inter
        iou = inter / jnp.maximum(union, 1e-8)
        row_i = lax.broadcasted_iota(jnp.int32, (32, _KPAD), 0) + c * 32
        s_scr[c * 32:(c + 1) * 32, :] = jnp.where(
            (iou > _THR) & (col_j > row_i), 1.0, 0.0)

    lane = lax.broadcasted_iota(jnp.int32, (1, _KPAD), 1)
    a = jnp.where(lane < _K, 1.0, 0.0)
    li = lax.broadcasted_iota(jnp.int32, (32, 32), 0)
    lj = lax.broadcasted_iota(jnp.int32, (32, 32), 1)
    eye = jnp.where(li == lj, 1.0, 0.0)
    for c in range(_KPAD // 32):
        base = c * 32
        slab = s_scr[base:base + 32, :]
        m_cc = slab[:, base:base + 32]
        m_t = lax.dot_general(eye, m_cc, (((1,), (1,)), ((), ())),
                              precision=lax.Precision.HIGHEST)
        a0_row = a[0:1, base:base + 32]
        a0_col = jnp.max(jnp.broadcast_to(a0_row, (32, 32)) * eye,
                         axis=1, keepdims=True)
        a_row, a_col = a0_row, a0_col
        for _ in range(34):
            sup_row = jnp.max(a_col * m_cc, axis=0, keepdims=True)
            sup_col = jnp.max(a_row * m_t, axis=1, keepdims=True)
            a_row = a0_row * (1.0 - sup_row)
            a_col = a0_col * (1.0 - sup_col)
        sup_g = jnp.max(a_col * slab, axis=0, keepdims=True)
        a = a * (1.0 - sup_g)

    ii = lax.broadcasted_iota(jnp.int32, (_KPAD, _KPAD), 0)
    jj = lax.broadcasted_iota(jnp.int32, (_KPAD, _KPAD), 1)
    tri = jnp.where(ii <= jj, 1.0, 0.0)
    rank = lax.dot_general(a, tri, (((1,), (0,)), ((), ()))) - 1.0
    rank_i = rank.astype(jnp.int32)

    rr = lax.broadcasted_iota(jnp.int32, (_MPAD, _KPAD), 0)
    sel = jnp.where((rr == rank_i) & (a > 0.5), 1.0, 0.0)
    o_ref[...] = lax.dot_general(sel, g_ref[...], (((1,), (0,)), ((), ())),
                                 precision=lax.Precision.HIGHEST)


def _nms_call(g, bT):
    return pl.pallas_call(
        _nms_body,
        out_shape=jax.ShapeDtypeStruct((_MPAD, _DPAD), jnp.float32),
        scratch_shapes=[pltpu.VMEM((_KPAD, _KPAD), jnp.float32)],
    )(g, bT)


def kernel(boxes, classification, detections):
    b = boxes[0]
    c = classification[0]
    d = detections[0]
    scores = _scores(c)
    _, idx = lax.top_k(scores, _K)
    idxp = jnp.concatenate(
        [idx.astype(jnp.int32), jnp.zeros((_KPAD - _K,), jnp.int32)])
    table = jnp.concatenate(
        [b, d, jnp.zeros((_N, _DPAD - 4 - _DDIM), jnp.float32)], axis=1)
    g = _make_sc_gather()(table, idxp)
    bT = jnp.pad(g[:, :4].T, ((0, 4), (0, 0)))
    res = _nms_call(g, bT)
    return res[None, :_MAXB, 4:4 + _DDIM]
